# Initial kernel scaffold; baseline (speedup 1.0000x reference)
import jax
import jax.numpy as jnp
from jax.experimental import pallas as pl


def kernel(x, w1f, w2p, g1, be1, g2, be2, w3p, b3p):
    raise NotImplementedError("write your pallas kernel here")


# speedup vs baseline: 1.3909x; 1.3909x over previous
import functools

import jax
import jax.numpy as jnp
from jax import lax
from jax.experimental import pallas as pl
from jax.experimental.pallas import tpu as pltpu

_VMEM_LIMIT = 64 * 1024 * 1024


def _conv1_kernel(xf_ref, rhs_ref, y1_ref, sc_ref, st_ref, *, Ho, Wo, cin_fp):
    xt = xf_ref[0]
    taps = [xt[a:a + Ho, b:b + Wo, :].reshape(Ho * Wo, cin_fp)
            for a in range(2) for b in range(2)]
    lhs = jnp.concatenate(taps, axis=1)
    acc = jnp.dot(lhs, rhs_ref[...], preferred_element_type=jnp.float32)
    y = acc[:, :128]
    s = jnp.sum(y, axis=0, keepdims=True)
    sq = jnp.sum(y * y, axis=0, keepdims=True)
    st_ref[0] = jnp.concatenate([s, sq], axis=0)
    sc_ref[0] = acc[:, 128:].astype(jnp.bfloat16)
    y1_ref[...] = jnp.zeros_like(y1_ref)
    y1_ref[0, 1:Ho + 1, 1:Wo + 1, :] = y.reshape(Ho, Wo, 128).astype(jnp.bfloat16)


def _conv2_kernel(y1_ref, rhs_ref, scale_ref, shift_ref, y2_ref, st_ref,
                  *, Ho, Wo):
    xt = y1_ref[0].astype(jnp.float32) * scale_ref[...] + shift_ref[...]
    xt = jnp.maximum(xt, 0.0)
    hi = lax.broadcasted_iota(jnp.int32, xt.shape, 0)
    wi = lax.broadcasted_iota(jnp.int32, xt.shape, 1)
    interior = (hi >= 1) & (hi <= Ho) & (wi >= 1) & (wi <= Wo)
    xtb = jnp.where(interior, xt, 0.0).astype(jnp.bfloat16)
    taps = [xtb[a:a + Ho, b:b + Wo, :].reshape(Ho * Wo, 128)
            for a in range(3) for b in range(3)]
    lhs = jnp.concatenate(taps, axis=1)
    acc = jnp.dot(lhs, rhs_ref[...], preferred_element_type=jnp.float32)
    s = jnp.sum(acc, axis=0, keepdims=True)
    sq = jnp.sum(acc * acc, axis=0, keepdims=True)
    st_ref[0] = jnp.concatenate([s, sq], axis=0)
    y2_ref[0] = acc.astype(jnp.bfloat16)


def _epilogue_kernel(y2_ref, sc_ref, scale_ref, shift_ref, b3_ref, o_ref):
    o_ref[...] = jnp.maximum(
        y2_ref[...].astype(jnp.float32) * scale_ref[...] + shift_ref[...]
        + sc_ref[...].astype(jnp.float32) + b3_ref[...], 0.0)


def _bn_fold(stats, count, gamma, beta, eps=1e-5):
    s = jnp.sum(stats[:, 0, :], axis=0)
    sq = jnp.sum(stats[:, 1, :], axis=0)
    mean = s / count
    var = jnp.maximum(sq / count - mean * mean, 0.0)
    scale = gamma * lax.rsqrt(var + eps)
    shift = beta - mean * scale
    return scale.reshape(1, -1), shift.reshape(1, -1)


def kernel(x, w1f, w2p, g1, be1, g2, be2, w3p, b3p):
    N, cin, H, W = x.shape
    s = 2
    Ho, Wo = (H + 2 - 3) // s + 1, (W + 2 - 3) // s + 1
    P = Ho * Wo
    M = N * P
    cin_fp = w1f.shape[1]
    cout_p = w1f.shape[2]

    xn = jnp.transpose(x, (0, 2, 3, 1))
    xp = jnp.pad(xn, ((0, 0), (1, 1), (1, 1), (0, 0)))
    Hf, Wf = (H + 2) // s, (W + 2) // s
    xf = xp.reshape(N, Hf, s, Wf, s, cin).transpose(0, 1, 3, 2, 4, 5)
    xf = xf.reshape(N, Hf, Wf, s * s * cin).astype(jnp.bfloat16)

    w3col = jnp.zeros((4 * cin_fp, cout_p), jnp.float32)
    w3col = w3col.at[3 * cin:3 * cin + cin].set(w3p[:cin])
    rhs1 = jnp.concatenate([w1f.reshape(4 * cin_fp, cout_p), w3col],
                           axis=1).astype(jnp.bfloat16)

    y1p, sc, st1 = pl.pallas_call(
        functools.partial(_conv1_kernel, Ho=Ho, Wo=Wo, cin_fp=cin_fp),
        out_shape=(jax.ShapeDtypeStruct((N, Ho + 2, Wo + 2, cout_p), jnp.bfloat16),
                   jax.ShapeDtypeStruct((N, P, cout_p), jnp.bfloat16),
                   jax.ShapeDtypeStruct((N, 2, cout_p), jnp.float32)),
        grid=(N,),
        in_specs=[pl.BlockSpec((1, Hf, Wf, cin_fp), lambda n: (n, 0, 0, 0)),
                  pl.BlockSpec((4 * cin_fp, 2 * cout_p), lambda n: (0, 0))],
        out_specs=(pl.BlockSpec((1, Ho + 2, Wo + 2, cout_p), lambda n: (n, 0, 0, 0)),
                   pl.BlockSpec((1, P, cout_p), lambda n: (n, 0, 0)),
                   pl.BlockSpec((1, 2, cout_p), lambda n: (n, 0, 0))),
        compiler_params=pltpu.CompilerParams(
            dimension_semantics=("parallel",),
            vmem_limit_bytes=_VMEM_LIMIT),
    )(xf, rhs1)

    scale1, shift1 = _bn_fold(st1, M, g1, be1)

    rhs2 = w2p.reshape(9 * cout_p, cout_p).astype(jnp.bfloat16)
    y2, st2 = pl.pallas_call(
        functools.partial(_conv2_kernel, Ho=Ho, Wo=Wo),
        out_shape=(jax.ShapeDtypeStruct((N, P, cout_p), jnp.bfloat16),
                   jax.ShapeDtypeStruct((N, 2, cout_p), jnp.float32)),
        grid=(N,),
        in_specs=[pl.BlockSpec((1, Ho + 2, Wo + 2, cout_p), lambda n: (n, 0, 0, 0)),
                  pl.BlockSpec((9 * cout_p, cout_p), lambda n: (0, 0)),
                  pl.BlockSpec((1, cout_p), lambda n: (0, 0)),
                  pl.BlockSpec((1, cout_p), lambda n: (0, 0))],
        out_specs=(pl.BlockSpec((1, P, cout_p), lambda n: (n, 0, 0)),
                   pl.BlockSpec((1, 2, cout_p), lambda n: (n, 0, 0))),
        compiler_params=pltpu.CompilerParams(
            dimension_semantics=("parallel",),
            vmem_limit_bytes=_VMEM_LIMIT),
    )(y1p, rhs2, scale1, shift1)

    scale2, shift2 = _bn_fold(st2, M, g2, be2)

    tm = next(t for t in (2048, 1024, P) if M % t == 0)
    chan = pl.BlockSpec((1, cout_p), lambda i: (0, 0))
    out = pl.pallas_call(
        _epilogue_kernel,
        out_shape=jax.ShapeDtypeStruct((M, cout_p), jnp.float32),
        grid=(M // tm,),
        in_specs=[pl.BlockSpec((tm, cout_p), lambda i: (i, 0)),
                  pl.BlockSpec((tm, cout_p), lambda i: (i, 0)),
                  chan, chan, chan],
        out_specs=pl.BlockSpec((tm, cout_p), lambda i: (i, 0)),
        compiler_params=pltpu.CompilerParams(
            dimension_semantics=("parallel",),
            vmem_limit_bytes=_VMEM_LIMIT),
    )(y2.reshape(M, cout_p), sc.reshape(M, cout_p), scale2, shift2, b3p)

    out = out.reshape(N, Ho, Wo, cout_p)
    return jnp.transpose(out, (0, 3, 1, 2))


# speedup vs baseline: 1.5524x; 1.1161x over previous
import functools

import jax
import jax.numpy as jnp
from jax import lax
from jax.experimental import pallas as pl
from jax.experimental.pallas import tpu as pltpu

_VMEM_LIMIT = 64 * 1024 * 1024


def _conv1_kernel(xf_ref, rhs_ref, y1_ref, sc_ref, st_ref):
    base = xf_ref[0]
    sh = base[1:929]
    lhs = jnp.concatenate(
        [base[0:896], sh[0:896], base[32:928], sh[32:928]], axis=1)
    acc = jnp.dot(lhs, rhs_ref[...], preferred_element_type=jnp.float32)
    ri = lax.broadcasted_iota(jnp.int32, (896, 1), 0)
    valid = (ri % 32) < 28
    y = jnp.where(valid, acc[:, :128], 0.0)
    st_ref[0] = jnp.concatenate(
        [jnp.sum(y, axis=0, keepdims=True),
         jnp.sum(y * y, axis=0, keepdims=True)], axis=0)
    sc_ref[0] = acc[:, 128:].astype(jnp.bfloat16)
    y1_ref[...] = jnp.zeros_like(y1_ref)
    y1_ref[0, 33:929, :] = y.astype(jnp.bfloat16)


def _conv2_kernel(y1_ref, rhs_ref, scale_ref, shift_ref, y2_ref, st_ref):
    yt = y1_ref[0].astype(jnp.float32) * scale_ref[...] + shift_ref[...]
    yt = jnp.maximum(yt, 0.0)
    ri = lax.broadcasted_iota(jnp.int32, (992, 1), 0)
    h = ri // 32
    w = ri % 32
    interior = (h >= 1) & (h <= 28) & (w >= 1) & (w <= 28)
    xtb = jnp.where(interior, yt, 0.0).astype(jnp.bfloat16)
    s1 = xtb[1:961]
    s2 = xtb[2:962]
    lhs = jnp.concatenate(
        [xtb[0:896], s1[0:896], s2[0:896],
         xtb[32:928], s1[32:928], s2[32:928],
         xtb[64:960], s1[64:960], s2[64:960]], axis=1)
    acc = jnp.dot(lhs, rhs_ref[...], preferred_element_type=jnp.float32)
    rj = lax.broadcasted_iota(jnp.int32, (896, 1), 0)
    ym = jnp.where((rj % 32) < 28, acc, 0.0)
    st_ref[0] = jnp.concatenate(
        [jnp.sum(ym, axis=0, keepdims=True),
         jnp.sum(ym * ym, axis=0, keepdims=True)], axis=0)
    y2_ref[0] = acc.astype(jnp.bfloat16)


def _epilogue_kernel(y2_ref, sc_ref, scale_ref, shift_ref, b3_ref, o_ref):
    o_ref[...] = jnp.maximum(
        y2_ref[...].astype(jnp.float32) * scale_ref[...] + shift_ref[...]
        + sc_ref[...].astype(jnp.float32) + b3_ref[...], 0.0)


def _bn_fold(stats, count, gamma, beta, eps=1e-5):
    s = jnp.sum(stats[:, 0, :], axis=0)
    sq = jnp.sum(stats[:, 1, :], axis=0)
    mean = s / count
    var = jnp.maximum(sq / count - mean * mean, 0.0)
    scale = gamma * lax.rsqrt(var + eps)
    shift = beta - mean * scale
    return scale.reshape(1, -1), shift.reshape(1, -1)


def kernel(x, w1f, w2p, g1, be1, g2, be2, w3p, b3p):
    N, cin, H, W = x.shape
    Ho, Wo = (H + 2 - 3) // 2 + 1, (W + 2 - 3) // 2 + 1
    M = N * Ho * Wo
    cin_fp = w1f.shape[1]
    cout_p = w1f.shape[2]

    xb = jnp.pad(x.astype(jnp.bfloat16), ((0, 0), (0, 0), (1, 3), (1, 7)))
    xf = xb.reshape(N, cin, 30, 2, 32, 2).transpose(0, 2, 4, 3, 5, 1)
    xf = xf.reshape(N, 960, 4 * cin)

    w3col = jnp.zeros((4 * cin_fp, cout_p), jnp.float32)
    w3col = w3col.at[3 * cin:3 * cin + cin].set(w3p[:cin])
    rhs1 = jnp.concatenate([w1f.reshape(4 * cin_fp, cout_p), w3col],
                           axis=1).astype(jnp.bfloat16)

    y1p, sc, st1 = pl.pallas_call(
        _conv1_kernel,
        out_shape=(jax.ShapeDtypeStruct((N, 992, cout_p), jnp.bfloat16),
                   jax.ShapeDtypeStruct((N, 896, cout_p), jnp.bfloat16),
                   jax.ShapeDtypeStruct((N, 2, cout_p), jnp.float32)),
        grid=(N,),
        in_specs=[pl.BlockSpec((1, 960, cin_fp), lambda n: (n, 0, 0)),
                  pl.BlockSpec((4 * cin_fp, 2 * cout_p), lambda n: (0, 0))],
        out_specs=(pl.BlockSpec((1, 992, cout_p), lambda n: (n, 0, 0)),
                   pl.BlockSpec((1, 896, cout_p), lambda n: (n, 0, 0)),
                   pl.BlockSpec((1, 2, cout_p), lambda n: (n, 0, 0))),
        compiler_params=pltpu.CompilerParams(
            dimension_semantics=("parallel",),
            vmem_limit_bytes=_VMEM_LIMIT),
    )(xf, rhs1)

    scale1, shift1 = _bn_fold(st1, M, g1, be1)

    rhs2 = w2p.reshape(9 * cout_p, cout_p).astype(jnp.bfloat16)
    y2, st2 = pl.pallas_call(
        _conv2_kernel,
        out_shape=(jax.ShapeDtypeStruct((N, 896, cout_p), jnp.bfloat16),
                   jax.ShapeDtypeStruct((N, 2, cout_p), jnp.float32)),
        grid=(N,),
        in_specs=[pl.BlockSpec((1, 992, cout_p), lambda n: (n, 0, 0)),
                  pl.BlockSpec((9 * cout_p, cout_p), lambda n: (0, 0)),
                  pl.BlockSpec((1, cout_p), lambda n: (0, 0)),
                  pl.BlockSpec((1, cout_p), lambda n: (0, 0))],
        out_specs=(pl.BlockSpec((1, 896, cout_p), lambda n: (n, 0, 0)),
                   pl.BlockSpec((1, 2, cout_p), lambda n: (n, 0, 0))),
        compiler_params=pltpu.CompilerParams(
            dimension_semantics=("parallel",),
            vmem_limit_bytes=_VMEM_LIMIT),
    )(y1p, rhs2, scale1, shift1)

    scale2, shift2 = _bn_fold(st2, M, g2, be2)

    Mg = N * 896
    tm = next(t for t in (2048, 896) if Mg % t == 0)
    chan = pl.BlockSpec((1, cout_p), lambda i: (0, 0))
    out = pl.pallas_call(
        _epilogue_kernel,
        out_shape=jax.ShapeDtypeStruct((Mg, cout_p), jnp.float32),
        grid=(Mg // tm,),
        in_specs=[pl.BlockSpec((tm, cout_p), lambda i: (i, 0)),
                  pl.BlockSpec((tm, cout_p), lambda i: (i, 0)),
                  chan, chan, chan],
        out_specs=pl.BlockSpec((tm, cout_p), lambda i: (i, 0)),
        compiler_params=pltpu.CompilerParams(
            dimension_semantics=("parallel",),
            vmem_limit_bytes=_VMEM_LIMIT),
    )(y2.reshape(Mg, cout_p), sc.reshape(Mg, cout_p), scale2, shift2, b3p)

    out = out.reshape(N, Ho, 32, cout_p)[:, :, :Wo, :]
    return jnp.transpose(out, (0, 3, 1, 2))


# speedup vs baseline: 1.6915x; 1.0896x over previous
import functools

import jax
import jax.numpy as jnp
from jax import lax
from jax.experimental import pallas as pl
from jax.experimental.pallas import tpu as pltpu

_VMEM_LIMIT = 64 * 1024 * 1024


def _conv1_kernel(xf_ref, rhs_ref, y1_ref, sc_ref, st_ref):
    v = xf_ref[0].reshape(30, 64, 128)
    hpe = v[:, 0:32, :].reshape(960, 128)
    hpo = v[:, 32:64, :].reshape(960, 128)
    she = hpe[1:929]
    sho = hpo[1:897]
    lhs = jnp.concatenate(
        [hpe[0:896], she[0:896], hpo[0:896], sho,
         hpe[32:928], she[32:928]], axis=1)
    acc = jnp.dot(lhs, rhs_ref[...], preferred_element_type=jnp.float32)
    ri = lax.broadcasted_iota(jnp.int32, (896, 1), 0)
    valid = (ri % 32) < 28
    y = jnp.where(valid, acc[:, :128], 0.0)
    st_ref[0] = jnp.concatenate(
        [jnp.sum(y, axis=0, keepdims=True),
         jnp.sum(y * y, axis=0, keepdims=True)], axis=0)
    sc_ref[0] = acc[:, 128:].astype(jnp.bfloat16)
    y1_ref[...] = jnp.zeros_like(y1_ref)
    y1_ref[0, 33:929, :] = y.astype(jnp.bfloat16)


def _conv2_kernel(y1_ref, rhs_ref, scale_ref, shift_ref, y2_ref, st_ref):
    yt = y1_ref[0].astype(jnp.float32) * scale_ref[...] + shift_ref[...]
    yt = jnp.maximum(yt, 0.0)
    ri = lax.broadcasted_iota(jnp.int32, (992, 1), 0)
    h = ri // 32
    w = ri % 32
    interior = (h >= 1) & (h <= 28) & (w >= 1) & (w <= 28)
    xtb = jnp.where(interior, yt, 0.0).astype(jnp.bfloat16)
    s1 = xtb[1:961]
    s2 = xtb[2:962]
    lhs = jnp.concatenate(
        [xtb[0:896], s1[0:896], s2[0:896],
         xtb[32:928], s1[32:928], s2[32:928],
         xtb[64:960], s1[64:960], s2[64:960]], axis=1)
    acc = jnp.dot(lhs, rhs_ref[...], preferred_element_type=jnp.float32)
    rj = lax.broadcasted_iota(jnp.int32, (896, 1), 0)
    ym = jnp.where((rj % 32) < 28, acc, 0.0)
    st_ref[0] = jnp.concatenate(
        [jnp.sum(ym, axis=0, keepdims=True),
         jnp.sum(ym * ym, axis=0, keepdims=True)], axis=0)
    y2_ref[0] = acc.astype(jnp.bfloat16)


def _epilogue_kernel(y2_ref, sc_ref, scale_ref, shift_ref, b3_ref, o_ref):
    o_ref[...] = jnp.maximum(
        y2_ref[...].astype(jnp.float32) * scale_ref[...] + shift_ref[...]
        + sc_ref[...].astype(jnp.float32) + b3_ref[...], 0.0)


def _bn_fold(stats, count, gamma, beta, eps=1e-5):
    s = jnp.sum(stats[:, 0, :], axis=0)
    sq = jnp.sum(stats[:, 1, :], axis=0)
    mean = s / count
    var = jnp.maximum(sq / count - mean * mean, 0.0)
    scale = gamma * lax.rsqrt(var + eps)
    shift = beta - mean * scale
    return scale.reshape(1, -1), shift.reshape(1, -1)


def kernel(x, w1f, w2p, g1, be1, g2, be2, w3p, b3p):
    N, cin, H, W = x.shape
    Ho, Wo = (H + 2 - 3) // 2 + 1, (W + 2 - 3) // 2 + 1
    M = N * Ho * Wo
    cin_fp = w1f.shape[1]
    cout_p = w1f.shape[2]

    xb = jnp.pad(jnp.transpose(x, (0, 2, 3, 1)).astype(jnp.bfloat16),
                 ((0, 0), (1, 3), (1, 7), (0, 0)))
    xf = xb.reshape(N, 60, 32, 2 * cin)

    def _wt(dy, dx):
        t = (dy // 2) * 2 + (dx // 2)
        slot = (dy % 2) * 2 + (dx % 2)
        return w1f[t, slot * cin:(slot + 1) * cin, :]
    blocks = []
    for dy in range(3):
        for b in range(2):
            top = _wt(dy, 2 * b)
            bot = _wt(dy, 2 * b + 1) if 2 * b + 1 < 3 else jnp.zeros_like(top)
            blocks.append(jnp.concatenate([top, bot], axis=0))
    w1cols = jnp.concatenate(blocks, axis=0)
    sccols = jnp.zeros((6 * 2 * cin, cout_p), jnp.float32)
    sccols = sccols.at[2 * 2 * cin + cin:2 * 2 * cin + 2 * cin].set(w3p[:cin])
    rhs1 = jnp.concatenate([w1cols, sccols], axis=1).astype(jnp.bfloat16)

    y1p, sc, st1 = pl.pallas_call(
        _conv1_kernel,
        out_shape=(jax.ShapeDtypeStruct((N, 992, cout_p), jnp.bfloat16),
                   jax.ShapeDtypeStruct((N, 896, cout_p), jnp.bfloat16),
                   jax.ShapeDtypeStruct((N, 2, cout_p), jnp.float32)),
        grid=(N,),
        in_specs=[pl.BlockSpec((1, 60, 32, 2 * cin), lambda n: (n, 0, 0, 0)),
                  pl.BlockSpec((6 * 2 * cin, 2 * cout_p), lambda n: (0, 0))],
        out_specs=(pl.BlockSpec((1, 992, cout_p), lambda n: (n, 0, 0)),
                   pl.BlockSpec((1, 896, cout_p), lambda n: (n, 0, 0)),
                   pl.BlockSpec((1, 2, cout_p), lambda n: (n, 0, 0))),
        compiler_params=pltpu.CompilerParams(
            dimension_semantics=("parallel",),
            vmem_limit_bytes=_VMEM_LIMIT),
    )(xf, rhs1)

    scale1, shift1 = _bn_fold(st1, M, g1, be1)

    rhs2 = w2p.reshape(9 * cout_p, cout_p).astype(jnp.bfloat16)
    y2, st2 = pl.pallas_call(
        _conv2_kernel,
        out_shape=(jax.ShapeDtypeStruct((N, 896, cout_p), jnp.bfloat16),
                   jax.ShapeDtypeStruct((N, 2, cout_p), jnp.float32)),
        grid=(N,),
        in_specs=[pl.BlockSpec((1, 992, cout_p), lambda n: (n, 0, 0)),
                  pl.BlockSpec((9 * cout_p, cout_p), lambda n: (0, 0)),
                  pl.BlockSpec((1, cout_p), lambda n: (0, 0)),
                  pl.BlockSpec((1, cout_p), lambda n: (0, 0))],
        out_specs=(pl.BlockSpec((1, 896, cout_p), lambda n: (n, 0, 0)),
                   pl.BlockSpec((1, 2, cout_p), lambda n: (n, 0, 0))),
        compiler_params=pltpu.CompilerParams(
            dimension_semantics=("parallel",),
            vmem_limit_bytes=_VMEM_LIMIT),
    )(y1p, rhs2, scale1, shift1)

    scale2, shift2 = _bn_fold(st2, M, g2, be2)

    Mg = N * 896
    tm = next(t for t in (2048, 896) if Mg % t == 0)
    chan = pl.BlockSpec((1, cout_p), lambda i: (0, 0))
    out = pl.pallas_call(
        _epilogue_kernel,
        out_shape=jax.ShapeDtypeStruct((Mg, cout_p), jnp.float32),
        grid=(Mg // tm,),
        in_specs=[pl.BlockSpec((tm, cout_p), lambda i: (i, 0)),
                  pl.BlockSpec((tm, cout_p), lambda i: (i, 0)),
                  chan, chan, chan],
        out_specs=pl.BlockSpec((tm, cout_p), lambda i: (i, 0)),
        compiler_params=pltpu.CompilerParams(
            dimension_semantics=("parallel",),
            vmem_limit_bytes=_VMEM_LIMIT),
    )(y2.reshape(Mg, cout_p), sc.reshape(Mg, cout_p), scale2, shift2, b3p)

    out = out.reshape(N, Ho, 32, cout_p)[:, :, :Wo, :]
    return jnp.transpose(out, (0, 3, 1, 2))
